# baseline (speedup 1.0000x reference)
import functools

import jax
import jax.numpy as jnp
from jax import lax
from jax.experimental import pallas as pl
from jax.experimental.pallas import tpu as pltpu
from jax.experimental.pallas import tpu_sc as plsc

_D = 256
_ROWS = 32 * 1024
_NC = 2
_NS = 16
_NW = _NC * _NS
_RPW = _ROWS // _NW
_CH = 128
_NCH = _RPW // _CH


@functools.partial(
    pl.kernel,
    mesh=plsc.VectorSubcoreMesh(core_axis_name="c", subcore_axis_name="s"),
    out_type=jax.ShapeDtypeStruct((_ROWS, _D), jnp.float32),
    scratch_types=[
        pltpu.VMEM((_CH, _D), jnp.float32),
    ],
)
def _sc_copy(x_hbm, o_hbm, buf):
    wid = lax.axis_index("s") * _NC + lax.axis_index("c")
    base = wid * _RPW
    for j in range(_NCH):
        pltpu.sync_copy(x_hbm.at[pl.ds(base + j * _CH, _CH)], buf)
        pltpu.sync_copy(buf, o_hbm.at[pl.ds(base + j * _CH, _CH)])


def kernel(x):
    return _sc_copy(x.reshape(-1, _D))


# speedup vs baseline: 1.1151x; 1.1151x over previous
import functools

import jax
import jax.numpy as jnp
from jax import lax
from jax.experimental import pallas as pl
from jax.experimental.pallas import tpu as pltpu
from jax.experimental.pallas import tpu_sc as plsc

_D = 256
_ROWS = 32 * 1024
_NC = 2
_NS = 16
_NW = _NC * _NS
_RPW = _ROWS // _NW
_CH = 64
_NCH = _RPW // _CH
_S = 4


@functools.partial(
    pl.kernel,
    mesh=plsc.VectorSubcoreMesh(core_axis_name="c", subcore_axis_name="s"),
    out_type=jax.ShapeDtypeStruct((_ROWS, _D), jnp.float32),
    scratch_types=[
        pltpu.VMEM((_S, _CH, _D), jnp.float32),
        pltpu.SemaphoreType.DMA((_S,)),
        pltpu.SemaphoreType.DMA((_S,)),
    ],
)
def _sc_copy(x_hbm, o_hbm, buf, rsems, wsems):
    wid = lax.axis_index("s") * _NC + lax.axis_index("c")
    base = wid * _RPW

    def read(j):
        s = j % _S
        return pltpu.make_async_copy(
            x_hbm.at[pl.ds(base + j * _CH, _CH)], buf.at[s], rsems.at[s])

    def write(j):
        s = j % _S
        return pltpu.make_async_copy(
            buf.at[s], o_hbm.at[pl.ds(base + j * _CH, _CH)], wsems.at[s])

    for j in range(_S):
        read(j).start()
    for j in range(_NCH):
        read(j).wait()
        write(j).start()
        if j + _S < _NCH:
            write(j).wait()
            read(j + _S).start()
    for j in range(_NCH - _S, _NCH):
        write(j).wait()


def kernel(x):
    return _sc_copy(x.reshape(-1, _D))
